# baseline (speedup 1.0000x reference)
import functools

import jax
import jax.numpy as jnp
import numpy as np
from jax import lax
from jax.experimental import pallas as pl
from jax.experimental.pallas import tpu as pltpu
from jax.experimental.pallas import tpu_sc as plsc

_NODES = 336
_NUM_CASE = 16
_TOTAL = 32768
_ACOUSTIC = float(np.sqrt(1.4 * 287 * 300))
_DENSITY = 1.225
_MIU = 1.9e-05
_ALPHA = float(6 * np.pi / 180)
_SPAN = 680
_CHUNKS = _NODES // 16


def _rsqrt(m):
    bits = plsc.bitcast(m, jnp.int32)
    y = plsc.bitcast(jnp.int32(0x5F3759DF) - (bits >> 1), jnp.float32)
    for _ in range(3):
        y = y * (1.5 - 0.5 * m * y * y)
    return y


def _sc_body(batch_hbm, cx_hbm, cy_hbm, f0_hbm, f2_hbm, f3_hbm, d3_hbm,
             d4_hbm, out_hbm, bbuf, offs_vmem, xbuf, ybuf, uxbuf, uybuf,
             ptbuf, d3buf, d4buf, taubuf, txbuf, tybuf, obuf, sbuf, o2buf,
             offs_shared, res_shared, sem):
    core = lax.axis_index("c")
    sub = lax.axis_index("s")
    lanes = lax.iota(jnp.int32, 16)

    @pl.when(jnp.logical_and(core == 0, sub == 0))
    def _():
        pltpu.sync_copy(batch_hbm, bbuf)
        lo0 = jnp.zeros((16,), jnp.int32)
        hi0 = jnp.full((16,), _TOTAL, jnp.int32)

        def step(_, carry):
            lo, hi = carry
            mid = (lo + hi) >> 1
            pred = plsc.load_gather(bbuf, [mid]) < lanes
            return jnp.where(pred, mid + 1, lo), jnp.where(pred, hi, mid)

        lo, _ = lax.fori_loop(0, 15, step, (lo0, hi0))
        offs_vmem[...] = lo
        pltpu.sync_copy(offs_vmem, offs_shared)

    plsc.subcore_barrier()

    @pl.when(core == 0)
    def _():
        pltpu.sync_copy(offs_shared, offs_vmem)
        case = sub
        offs = jnp.sum(jnp.where(lanes == case, offs_vmem[...], 0))
        offs = jnp.minimum(offs, _TOTAL - 2 * _NODES)
        base = jnp.minimum(offs, _TOTAL - _SPAN)
        s8 = pl.multiple_of((base // 8) * 8, 8)
        delta = offs - s8

        copies = [
            pltpu.async_copy(cx_hbm.at[pl.ds(s8, _SPAN)], xbuf, sem),
            pltpu.async_copy(cy_hbm.at[pl.ds(s8, _SPAN)], ybuf, sem),
            pltpu.async_copy(f2_hbm.at[pl.ds(s8, _SPAN)], uxbuf, sem),
            pltpu.async_copy(f3_hbm.at[pl.ds(s8, _SPAN)], uybuf, sem),
            pltpu.async_copy(f0_hbm.at[pl.ds(s8, _SPAN)], ptbuf, sem),
            pltpu.async_copy(d3_hbm.at[pl.ds(s8, _SPAN)], d3buf, sem),
            pltpu.async_copy(d4_hbm.at[pl.ds(s8, _SPAN)], d4buf, sem),
        ]
        for c in copies:
            c.wait()

        zero = jnp.zeros((16,), jnp.float32)

        def pass_a(g, acc):
            a_px, a_py, a_d3, a_d4 = acc
            i = g * 16 + lanes
            inext = jnp.where(i == _NODES - 1, 0, i + 1)
            r0 = delta + i
            rn = delta + inext
            r1 = delta + _NODES + i
            x0 = plsc.load_gather(xbuf, [r0])
            y0 = plsc.load_gather(ybuf, [r0])
            xn = plsc.load_gather(xbuf, [rn])
            yn = plsc.load_gather(ybuf, [rn])
            x1 = plsc.load_gather(xbuf, [r1])
            y1 = plsc.load_gather(ybuf, [r1])
            pt = plsc.load_gather(ptbuf, [r0])
            ux = plsc.load_gather(uxbuf, [r1])
            uy = plsc.load_gather(uybuf, [r1])
            d3 = plsc.load_gather(d3buf, [r0])
            d4 = plsc.load_gather(d4buf, [r0])
            tx = xn - x0
            ty = yn - y0
            dx = x1 - x0
            dy = y1 - y0
            m = (tx * tx + ty * ty) * (dx * dx + dy * dy)
            tau = _MIU * (ux * tx + uy * ty) * _rsqrt(m)
            taubuf[pl.ds(g * 16, 16)] = tau
            txbuf[pl.ds(g * 16, 16)] = tx
            tybuf[pl.ds(g * 16, 16)] = ty
            return (a_px + pt * ty, a_py + pt * tx, a_d3 + d3, a_d4 + d4)

        a_px, a_py, a_d3, a_d4 = lax.fori_loop(
            0, _CHUNKS, pass_a, (zero, zero, zero, zero))

        def pass_b(g, acc):
            b_tx, b_ty = acc
            i = g * 16 + lanes
            inext = jnp.where(i == _NODES - 1, 0, i + 1)
            tau_i = taubuf[pl.ds(g * 16, 16)]
            tau_n = plsc.load_gather(taubuf, [inext])
            tx = txbuf[pl.ds(g * 16, 16)]
            ty = tybuf[pl.ds(g * 16, 16)]
            ta = (tau_i + tau_n) * 0.5
            return (b_tx + ta * tx, b_ty + ta * ty)

        b_tx, b_ty = lax.fori_loop(0, _CHUNKS, pass_b, (zero, zero))

        fx = 50.0 * jnp.sum(b_tx) - jnp.sum(a_px)
        fy = 50.0 * jnp.sum(b_ty) + jnp.sum(a_py)
        ma = jnp.sum(a_d3) * (0.3 / _NODES) + 0.3
        af = jnp.sum(a_d4) * (_ALPHA / _NODES)
        a2 = af * af
        ca = 1.0 + a2 * (-0.5 + a2 * (1.0 / 24.0 + a2 * (-1.0 / 720.0)))
        sa = af * (1.0 + a2 * (-1.0 / 6.0 + a2 * (1.0 / 120.0
                                                  + a2 * (-1.0 / 5040.0))))
        fx2 = fx * ca + fy * sa
        fy2 = fy * ca - fx2 * sa
        vel = _ACOUSTIC * ma
        q = 0.5 * _DENSITY * vel * vel
        num = jnp.where(lanes == 0, fx2, jnp.where(lanes == 1, fy2, 0.0))
        res = num / q
        obuf[...] = res
        pltpu.sync_copy(obuf, res_shared.at[case])

    plsc.subcore_barrier()

    @pl.when(jnp.logical_and(core == 0, sub == 0))
    def _():
        pltpu.sync_copy(res_shared, sbuf)
        col0 = jnp.zeros((16,), jnp.int32)
        col1 = jnp.full((16,), 1, jnp.int32)
        cd = plsc.load_gather(sbuf, [lanes, col0])
        cl = plsc.load_gather(sbuf, [lanes, col1])
        plsc.store_scatter(o2buf, [lanes, col0], cd)
        plsc.store_scatter(o2buf, [lanes, col1], cl)
        pltpu.sync_copy(o2buf, out_hbm)


_sc_dynamicor = functools.partial(
    pl.kernel,
    out_type=jax.ShapeDtypeStruct((_NUM_CASE, 2), jnp.float32),
    mesh=plsc.VectorSubcoreMesh(
        core_axis_name="c", subcore_axis_name="s", num_cores=1),
    compiler_params=pltpu.CompilerParams(
        needs_layout_passes=False, use_tc_tiling_on_sc=False),
    scratch_types=[
        pltpu.VMEM((_TOTAL,), jnp.int32),
        pltpu.VMEM((16,), jnp.int32),
        pltpu.VMEM((_SPAN,), jnp.float32),
        pltpu.VMEM((_SPAN,), jnp.float32),
        pltpu.VMEM((_SPAN,), jnp.float32),
        pltpu.VMEM((_SPAN,), jnp.float32),
        pltpu.VMEM((_SPAN,), jnp.float32),
        pltpu.VMEM((_SPAN,), jnp.float32),
        pltpu.VMEM((_SPAN,), jnp.float32),
        pltpu.VMEM((_NODES,), jnp.float32),
        pltpu.VMEM((_NODES,), jnp.float32),
        pltpu.VMEM((_NODES,), jnp.float32),
        pltpu.VMEM((16,), jnp.float32),
        pltpu.VMEM((16, 16), jnp.float32),
        pltpu.VMEM((16, 2), jnp.float32),
        pltpu.VMEM_SHARED((16,), jnp.int32),
        pltpu.VMEM_SHARED((16, 16), jnp.float32),
        pltpu.SemaphoreType.DMA,
    ],
)(_sc_body)


def kernel(batch, coords, fields, design):
    return _sc_dynamicor(
        batch.astype(jnp.int32),
        coords[:, 0], coords[:, 1],
        fields[:, 0], fields[:, 2], fields[:, 3],
        design[:, 3], design[:, 4],
    )


# speedup vs baseline: 1.0529x; 1.0529x over previous
import functools

import jax
import jax.numpy as jnp
import numpy as np
from jax import lax
from jax.experimental import pallas as pl
from jax.experimental.pallas import tpu as pltpu
from jax.experimental.pallas import tpu_sc as plsc

_NODES = 336
_NUM_CASE = 16
_TOTAL = 32768
_ACOUSTIC = float(np.sqrt(1.4 * 287 * 300))
_DENSITY = 1.225
_MIU = 1.9e-05
_ALPHA = float(6 * np.pi / 180)
_SPAN = 680
_CHUNKS = _NODES // 16


def _rsqrt(m):
    bits = plsc.bitcast(m, jnp.int32)
    y = plsc.bitcast(jnp.int32(0x5F3759DF) - (bits >> 1), jnp.float32)
    for _ in range(3):
        y = y * (1.5 - 0.5 * m * y * y)
    return y


_CHUNK = _TOTAL // 16


def _sc_body(batch_hbm, cx_hbm, cy_hbm, f0_hbm, f2_hbm, f3_hbm, d3_hbm,
             d4_hbm, out_hbm, bbuf, offs_vmem, cntbuf, xbuf, ybuf, uxbuf,
             uybuf, ptbuf, d3buf, d4buf, taubuf, txbuf, tybuf, obuf,
             cnt_shared, sem):
    core = lax.axis_index("c")
    sub = lax.axis_index("s")
    lanes = lax.iota(jnp.int32, 16)

    b8 = pl.multiple_of(sub * _CHUNK, 8)
    pltpu.sync_copy(batch_hbm.at[pl.ds(b8, _CHUNK)], bbuf)
    lo0 = jnp.zeros((16,), jnp.int32)
    hi0 = jnp.full((16,), _CHUNK, jnp.int32)

    def step(_, carry):
        lo, hi = carry
        mid = (lo + hi) >> 1
        pred = plsc.load_gather(bbuf, [mid]) < lanes
        return jnp.where(pred, mid + 1, lo), jnp.where(pred, hi, mid)

    lo, _ = lax.fori_loop(0, 11, step, (lo0, hi0))
    offs_vmem[...] = lo
    pltpu.sync_copy(offs_vmem, cnt_shared.at[sub])

    plsc.subcore_barrier()

    @pl.when(core == 0)
    def _():
        pltpu.sync_copy(cnt_shared, cntbuf)
        offs_vec = cntbuf[0, :]
        for t in range(1, 16):
            offs_vec = offs_vec + cntbuf[t, :]
        case = sub
        offs = jnp.sum(jnp.where(lanes == case, offs_vec, 0))
        offs = jnp.minimum(offs, _TOTAL - 2 * _NODES)
        base = jnp.minimum(offs, _TOTAL - _SPAN)
        s8 = pl.multiple_of((base // 8) * 8, 8)
        delta = offs - s8

        copies = [
            pltpu.async_copy(cx_hbm.at[pl.ds(s8, _SPAN)], xbuf, sem),
            pltpu.async_copy(cy_hbm.at[pl.ds(s8, _SPAN)], ybuf, sem),
            pltpu.async_copy(f2_hbm.at[pl.ds(s8, _SPAN)], uxbuf, sem),
            pltpu.async_copy(f3_hbm.at[pl.ds(s8, _SPAN)], uybuf, sem),
            pltpu.async_copy(f0_hbm.at[pl.ds(s8, _SPAN)], ptbuf, sem),
            pltpu.async_copy(d3_hbm.at[pl.ds(s8, _SPAN)], d3buf, sem),
            pltpu.async_copy(d4_hbm.at[pl.ds(s8, _SPAN)], d4buf, sem),
        ]
        for c in copies:
            c.wait()

        zero = jnp.zeros((16,), jnp.float32)

        def pass_a(g, acc):
            a_px, a_py, a_d3, a_d4 = acc
            i = g * 16 + lanes
            inext = jnp.where(i == _NODES - 1, 0, i + 1)
            r0 = delta + i
            rn = delta + inext
            r1 = delta + _NODES + i
            x0 = plsc.load_gather(xbuf, [r0])
            y0 = plsc.load_gather(ybuf, [r0])
            xn = plsc.load_gather(xbuf, [rn])
            yn = plsc.load_gather(ybuf, [rn])
            x1 = plsc.load_gather(xbuf, [r1])
            y1 = plsc.load_gather(ybuf, [r1])
            pt = plsc.load_gather(ptbuf, [r0])
            ux = plsc.load_gather(uxbuf, [r1])
            uy = plsc.load_gather(uybuf, [r1])
            d3 = plsc.load_gather(d3buf, [r0])
            d4 = plsc.load_gather(d4buf, [r0])
            tx = xn - x0
            ty = yn - y0
            dx = x1 - x0
            dy = y1 - y0
            m = (tx * tx + ty * ty) * (dx * dx + dy * dy)
            tau = _MIU * (ux * tx + uy * ty) * _rsqrt(m)
            taubuf[pl.ds(g * 16, 16)] = tau
            txbuf[pl.ds(g * 16, 16)] = tx
            tybuf[pl.ds(g * 16, 16)] = ty
            return (a_px + pt * ty, a_py + pt * tx, a_d3 + d3, a_d4 + d4)

        a_px, a_py, a_d3, a_d4 = lax.fori_loop(
            0, _CHUNKS, pass_a, (zero, zero, zero, zero))

        def pass_b(g, acc):
            b_tx, b_ty = acc
            i = g * 16 + lanes
            inext = jnp.where(i == _NODES - 1, 0, i + 1)
            tau_i = taubuf[pl.ds(g * 16, 16)]
            tau_n = plsc.load_gather(taubuf, [inext])
            tx = txbuf[pl.ds(g * 16, 16)]
            ty = tybuf[pl.ds(g * 16, 16)]
            ta = (tau_i + tau_n) * 0.5
            return (b_tx + ta * tx, b_ty + ta * ty)

        b_tx, b_ty = lax.fori_loop(0, _CHUNKS, pass_b, (zero, zero))

        fx = 50.0 * jnp.sum(b_tx) - jnp.sum(a_px)
        fy = 50.0 * jnp.sum(b_ty) + jnp.sum(a_py)
        ma = jnp.sum(a_d3) * (0.3 / _NODES) + 0.3
        af = jnp.sum(a_d4) * (_ALPHA / _NODES)
        a2 = af * af
        ca = 1.0 + a2 * (-0.5 + a2 * (1.0 / 24.0 + a2 * (-1.0 / 720.0)))
        sa = af * (1.0 + a2 * (-1.0 / 6.0 + a2 * (1.0 / 120.0
                                                  + a2 * (-1.0 / 5040.0))))
        fx2 = fx * ca + fy * sa
        fy2 = fy * ca - fx2 * sa
        vel = _ACOUSTIC * ma
        q = 0.5 * _DENSITY * vel * vel
        num = jnp.where(lanes == 0, fx2, jnp.where(lanes == 1, fy2, 0.0))
        res = num / q
        obuf[...] = res
        pltpu.sync_copy(obuf, out_hbm.at[case])


_sc_dynamicor = functools.partial(
    pl.kernel,
    out_type=jax.ShapeDtypeStruct((_NUM_CASE, 16), jnp.float32),
    mesh=plsc.VectorSubcoreMesh(
        core_axis_name="c", subcore_axis_name="s", num_cores=1),
    compiler_params=pltpu.CompilerParams(
        needs_layout_passes=False, use_tc_tiling_on_sc=False),
    scratch_types=[
        pltpu.VMEM((_CHUNK,), jnp.int32),
        pltpu.VMEM((16,), jnp.int32),
        pltpu.VMEM((16, 16), jnp.int32),
        pltpu.VMEM((_SPAN,), jnp.float32),
        pltpu.VMEM((_SPAN,), jnp.float32),
        pltpu.VMEM((_SPAN,), jnp.float32),
        pltpu.VMEM((_SPAN,), jnp.float32),
        pltpu.VMEM((_SPAN,), jnp.float32),
        pltpu.VMEM((_SPAN,), jnp.float32),
        pltpu.VMEM((_SPAN,), jnp.float32),
        pltpu.VMEM((_NODES,), jnp.float32),
        pltpu.VMEM((_NODES,), jnp.float32),
        pltpu.VMEM((_NODES,), jnp.float32),
        pltpu.VMEM((16,), jnp.float32),
        pltpu.VMEM_SHARED((16, 16), jnp.int32),
        pltpu.SemaphoreType.DMA,
    ],
)(_sc_body)


def kernel(batch, coords, fields, design):
    out = _sc_dynamicor(
        batch.astype(jnp.int32),
        coords[:, 0], coords[:, 1],
        fields[:, 0], fields[:, 2], fields[:, 3],
        design[:, 3], design[:, 4],
    )
    return out[:, :2]
